# baseline (speedup 1.0000x reference)
import functools

import jax
import jax.numpy as jnp
from jax import lax
from jax.experimental import pallas as pl
from jax.experimental.pallas import tpu as pltpu
from jax.experimental.pallas import tpu_sc as plsc

NUM_EMB = 1000000
DIM = 64
BATCH = 4096
SEQ = 200

CHUNK = 128
TOTAL = BATCH * SEQ
TOTAL_CHUNKS = TOTAL // CHUNK
NW = 32
CHUNKS_PER_W = TOTAL_CHUNKS // NW
NBUF = 4


def _gather_sc(tok2d, table):
    mesh = plsc.VectorSubcoreMesh(core_axis_name="c", subcore_axis_name="s")

    @functools.partial(
        pl.kernel,
        mesh=mesh,
        out_type=jax.ShapeDtypeStruct((TOTAL, DIM), jnp.float32),
        compiler_params=pltpu.CompilerParams(use_tc_tiling_on_sc=False),
        scratch_types=(
            [pltpu.VMEM((CHUNKS_PER_W, CHUNK), jnp.int32)]
            + [pltpu.VMEM((CHUNK, DIM), jnp.float32) for _ in range(NBUF)]
            + [pltpu.SemaphoreType.DMA for _ in range(NBUF)]
        ),
    )
    def body(tok_hbm, table_hbm, out_hbm, idx_v, *rest):
        bufs = rest[:NBUF]
        sems = rest[NBUF:]
        wid = lax.axis_index("s") * 2 + lax.axis_index("c")
        row0 = wid * CHUNKS_PER_W
        obase = row0 * CHUNK

        pltpu.sync_copy(tok_hbm.at[pl.ds(row0, CHUNKS_PER_W)], idx_v)

        for b in range(NBUF):
            pltpu.async_copy(table_hbm.at[idx_v.at[b]], bufs[b], sems[b])

        def outer(i, carry):
            g = i * NBUF
            for b in range(NBUF):
                j = g + b
                pltpu.make_async_copy(
                    table_hbm.at[idx_v.at[b]], bufs[b], sems[b]
                ).wait()
                pltpu.sync_copy(
                    bufs[b], out_hbm.at[pl.ds(obase + j * CHUNK, CHUNK)]
                )

                @pl.when(j + NBUF < CHUNKS_PER_W)
                def _():
                    pltpu.async_copy(
                        table_hbm.at[idx_v.at[j + NBUF]], bufs[b], sems[b]
                    )

            return carry

        lax.fori_loop(0, CHUNKS_PER_W // NBUF, outer, 0)

    return body(tok2d, table)


def kernel(token_ids, embeddings):
    tok2d = token_ids.astype(jnp.int32).reshape(TOTAL_CHUNKS, CHUNK)
    out = _gather_sc(tok2d, embeddings)
    return out.reshape(token_ids.shape + (DIM,))
